# baseline (speedup 1.0000x reference)
import functools

import jax
import jax.numpy as jnp
from jax import lax
from jax.experimental import pallas as pl
from jax.experimental.pallas import tpu as pltpu
from jax.experimental.pallas import tpu_sc as plsc

_BATCH = 16384
_EMB_DIM = 32
_NC = 2
_NS = 16
_NW = _NC * _NS
_B_PER_W = _BATCH // _NW


def kernel(condition, embedding_weight):
    mesh = plsc.VectorSubcoreMesh(core_axis_name="c", subcore_axis_name="s")
    wt = embedding_weight.T
    idx = condition.astype(jnp.int32)

    @functools.partial(
        pl.kernel,
        mesh=mesh,
        out_type=jax.ShapeDtypeStruct((_EMB_DIM, _BATCH), jnp.float32),
        scratch_types=[
            pltpu.VMEM((_B_PER_W,), jnp.int32),
            pltpu.VMEM((_EMB_DIM, _B_PER_W), jnp.float32),
            pltpu.SemaphoreType.DMA,
        ],
        compiler_params=pltpu.CompilerParams(use_tc_tiling_on_sc=False),
    )
    def k(table_hbm, idx_hbm, out_hbm, idx_v, col_v, sem):
        wid = lax.axis_index("s") * _NC + lax.axis_index("c")
        base = wid * _B_PER_W
        pltpu.sync_copy(idx_hbm.at[pl.ds(base, _B_PER_W)], idx_v)

        for d in range(_EMB_DIM):
            pltpu.async_copy(
                table_hbm.at[d].at[idx_v], col_v.at[d], sem
            )

        pltpu.make_async_copy(
            table_hbm.at[0].at[pl.ds(0, _B_PER_W)], col_v.at[0], sem
        ).wait()
        for _ in range(_EMB_DIM - 1):
            pltpu.make_async_copy(
                table_hbm.at[0].at[pl.ds(0, _B_PER_W)], col_v.at[0], sem
            ).wait()

        pltpu.sync_copy(col_v, out_hbm.at[:, pl.ds(base, _B_PER_W)])

    outT = k(wt, idx)
    return outT.T


# speedup vs baseline: 3.2222x; 3.2222x over previous
import functools

import jax
import jax.numpy as jnp
from jax import lax
from jax.experimental import pallas as pl
from jax.experimental.pallas import tpu as pltpu
from jax.experimental.pallas import tpu_sc as plsc

_BATCH = 16384
_EMB_DIM = 32
_NC = 2
_NS = 16
_NW = _NC * _NS
_B_PER_W = _BATCH // _NW
_NTILE_FULL = 7812
_TAIL_R0 = _NTILE_FULL * 128
_HALF = _B_PER_W // 2


def _mesh():
    return plsc.VectorSubcoreMesh(core_axis_name="c", subcore_axis_name="s")


def _relayout(wt, tail4):
    @functools.partial(
        pl.kernel,
        mesh=_mesh(),
        out_type=jax.ShapeDtypeStruct((250000, 128), jnp.float32),
        scratch_types=[
            pltpu.VMEM((2, _EMB_DIM, 128), jnp.float32),
            pltpu.VMEM((2, 32, 128), jnp.float32),
            pltpu.SemaphoreType.DMA((2,)),
            pltpu.SemaphoreType.DMA((2,)),
        ],
        compiler_params=pltpu.CompilerParams(needs_layout_passes=False),
    )
    def k(wt_hbm, tail_hbm, t4_hbm, xb, ob, insem, outsem):
        wid = lax.axis_index("s") * _NC + lax.axis_index("c")
        nk = (_NTILE_FULL + 31 - wid) // 32

        iota = lax.iota(jnp.int32, 16)
        idx0_even = iota
        idx0_odd = iota + 16

        def start_in(i, slot):
            r0 = (wid + 32 * i) * 128
            pltpu.async_copy(
                wt_hbm.at[:, pl.ds(r0, 128)], xb.at[slot], insem.at[slot]
            )

        def wait_in(slot):
            pltpu.make_async_copy(
                wt_hbm.at[:, pl.ds(0, 128)], xb.at[slot], insem.at[slot]
            ).wait()

        def start_out(i, slot):
            row0 = (wid + 32 * i) * 32
            pltpu.async_copy(
                ob.at[slot], t4_hbm.at[pl.ds(row0, 32)], outsem.at[slot]
            )

        def wait_out(slot):
            pltpu.make_async_copy(
                ob.at[slot], t4_hbm.at[pl.ds(0, 32)], outsem.at[slot]
            ).wait()

        def permute(slot):
            @pl.loop(0, 32)
            def _(p):
                base = jnp.full((16,), 4 * p, jnp.int32)
                for qr in range(8):
                    idx0 = idx0_odd if qr & 1 else idx0_even
                    idx1 = base + (qr // 2)
                    val = plsc.load_gather(xb.at[slot], [idx0, idx1])
                    ob[slot, p, pl.ds(16 * qr, 16)] = val

        start_in(0, 0)

        @pl.loop(0, 123)
        def _(j):
            for b in range(2):
                i = 2 * j + b

                @pl.when(i < nk)
                def _():
                    @pl.when(i + 1 < nk)
                    def _():
                        start_in(i + 1, 1 - b)

                    wait_in(b)

                    @pl.when(i >= 2)
                    def _():
                        wait_out(b)

                    permute(b)
                    start_out(i, b)

        wait_out(0)
        wait_out(1)

        @pl.when(wid == 4)
        def _():
            pltpu.sync_copy(tail_hbm, xb.at[0, pl.ds(0, 16), :])
            pltpu.sync_copy(
                xb.at[0, pl.ds(0, 16), :], t4_hbm.at[pl.ds(249984, 16)]
            )

    return k(wt, tail4)


def _gather(t4, idx):
    @functools.partial(
        pl.kernel,
        mesh=_mesh(),
        out_type=jax.ShapeDtypeStruct((_BATCH, _EMB_DIM), jnp.float32),
        scratch_types=[
            pltpu.VMEM((_B_PER_W,), jnp.int32),
            pltpu.VMEM((_HALF,), jnp.int32),
            pltpu.VMEM((_HALF, 128), jnp.float32),
            pltpu.VMEM((_HALF, _EMB_DIM), jnp.float32),
            pltpu.SemaphoreType.DMA,
        ],
    )
    def k(t4_hbm, idx_hbm, out_hbm, idx_v, q_v, rows_v, out_v, sem):
        wid = lax.axis_index("s") * _NC + lax.axis_index("c")
        base = wid * _B_PER_W
        pltpu.sync_copy(idx_hbm.at[pl.ds(base, _B_PER_W)], idx_v)

        for h in range(2):
            h0 = h * _HALF

            @pl.loop(0, _HALF, step=16)
            def _(i0):
                q_v[pl.ds(i0, 16)] = idx_v[pl.ds(h0 + i0, 16)] >> 2

            pltpu.async_copy(t4_hbm.at[q_v], rows_v, sem).wait()

            @pl.loop(0, _HALF, step=16)
            def _(j0):
                v = idx_v[pl.ds(h0 + j0, 16)]
                for t in range(16):
                    off = (v[t] & 3) * _EMB_DIM
                    out_v[j0 + t, pl.ds(0, 16)] = rows_v[
                        j0 + t, pl.ds(off, 16)
                    ]
                    out_v[j0 + t, pl.ds(16, 16)] = rows_v[
                        j0 + t, pl.ds(off + 16, 16)
                    ]

            pltpu.sync_copy(out_v, out_hbm.at[pl.ds(base + h0, _HALF)])

    return k(t4, idx)


def kernel(condition, embedding_weight):
    wt = embedding_weight.T
    tail4 = embedding_weight[_TAIL_R0:, :].reshape(16, 128)
    idx = condition.astype(jnp.int32)
    t4 = _relayout(wt, tail4)
    return _gather(t4, idx)


# speedup vs baseline: 8.1513x; 2.5298x over previous
import functools

import jax
import jax.numpy as jnp
from jax import lax
from jax.experimental import pallas as pl
from jax.experimental.pallas import tpu as pltpu
from jax.experimental.pallas import tpu_sc as plsc

_BATCH = 16384
_EMB_DIM = 32
_NC = 2
_NS = 16
_NW = _NC * _NS
_B_PER_W = _BATCH // _NW


def kernel(condition, embedding_weight):
    mesh = plsc.VectorSubcoreMesh(core_axis_name="c", subcore_axis_name="s")

    @functools.partial(
        pl.kernel,
        mesh=mesh,
        out_type=jax.ShapeDtypeStruct((_BATCH, _EMB_DIM), jnp.float32),
        scratch_types=[
            pltpu.VMEM((_B_PER_W,), jnp.int32),
            pltpu.VMEM((_B_PER_W, _EMB_DIM), jnp.float32),
            pltpu.SemaphoreType.DMA,
        ],
    )
    def k(table_hbm, idx_hbm, out_hbm, idx_v, rows_v, sem):
        wid = lax.axis_index("s") * _NC + lax.axis_index("c")
        base = wid * _B_PER_W
        pltpu.sync_copy(idx_hbm.at[pl.ds(base, _B_PER_W)], idx_v)

        @pl.loop(0, _B_PER_W, step=16)
        def _(j0):
            v = idx_v[pl.ds(j0, 16)]
            for t in range(16):
                pltpu.async_copy(table_hbm.at[v[t]], rows_v.at[j0 + t], sem)

        pltpu.make_async_copy(
            table_hbm.at[pl.ds(0, _B_PER_W)], rows_v, sem
        ).wait()
        pltpu.sync_copy(rows_v, out_hbm.at[pl.ds(base, _B_PER_W)])

    return k(embedding_weight, condition.astype(jnp.int32))
